# baseline (speedup 1.0000x reference)
import functools

import jax
import jax.numpy as jnp
from jax import lax
from jax.experimental import pallas as pl
from jax.experimental.pallas import tpu as pltpu
from jax.experimental.pallas import tpu_sc as plsc

NC = 2
NS = 16
NW = NC * NS
L = 16

B, S = 16384, 200
N = B * S
PER_W = N // NW
CH = 6400
NCHUNK = PER_W // CH


def _body(x_hbm, nums_hbm, m_hbm, tab_hbm, c6_hbm, al_hbm,
          x_v0, x_v1, nums_v0, nums_v1, m_v0, m_v1, tab_v,
          oc6_v0, oc6_v1, oal_v0, oal_v1,
          sem_in0, sem_in1, sem_out0, sem_out1):
    wid = lax.axis_index("s") * NC + lax.axis_index("c")
    base = wid * PER_W
    x_v = (x_v0, x_v1)
    nums_v = (nums_v0, nums_v1)
    m_v = (m_v0, m_v1)
    oc6_v = (oc6_v0, oc6_v1)
    oal_v = (oal_v0, oal_v1)
    sem_in = (sem_in0, sem_in1)
    sem_out = (sem_out0, sem_out1)

    pltpu.sync_copy(tab_hbm, tab_v)

    iota = lax.iota(jnp.int32, L)
    two_iota = iota * 2
    byte_perm = iota >> 2
    byte_shift = (iota & 3) * 8

    def load_chunk(c, slot):
        ebase = pl.multiple_of(base + c * CH, 64)
        return (
            pltpu.async_copy(x_hbm.at[pl.ds(ebase * 2, CH * 2)], x_v[slot],
                             sem_in[slot]),
            pltpu.async_copy(nums_hbm.at[pl.ds(ebase, CH)], nums_v[slot],
                             sem_in[slot]),
            pltpu.async_copy(
                m_hbm.at[pl.ds(pl.multiple_of(ebase // 4, 16), CH // 4)],
                m_v[slot], sem_in[slot]),
        )

    def compute_chunk(c, slot):
        xs, ns, ms = x_v[slot], nums_v[slot], m_v[slot]
        oc6, oal = oc6_v[slot], oal_v[slot]

        def group(g, _):
            for k in range(4):
                t = g * 4 + k
                mbytes = plsc.load_gather(ms, [byte_perm + 4 * t])
                mask_k = (mbytes >> byte_shift) & 1
                nums16 = ns[pl.ds(t * L, L)]
                idx = jnp.where(mask_k != 0, 128, nums16 * 2)
                c6t = plsc.load_gather(tab_v, [idx])
                alt = plsc.load_gather(tab_v, [idx + 1])
                xidx = two_iota + t * (2 * L)
                c6x = plsc.load_gather(xs, [xidx])
                alx = plsc.load_gather(xs, [xidx + 1])
                oc6[pl.ds(t * L, L)] = c6t * jnp.exp(
                    jnp.clip(c6x, -4.0, 4.0))
                oal[pl.ds(t * L, L)] = alt * jnp.exp(
                    jnp.clip(alx, -4.0, 4.0))
            return _

        lax.fori_loop(0, CH // 64, group, None)
        ebase = pl.multiple_of(base + c * CH, 64)
        return (
            pltpu.async_copy(oc6, c6_hbm.at[pl.ds(ebase, CH)], sem_out[slot]),
            pltpu.async_copy(oal, al_hbm.at[pl.ds(ebase, CH)], sem_out[slot]),
        )

    in_flight = load_chunk(0, 0)
    out_flight = (None, None)
    for c in range(NCHUNK):
        slot = c % 2
        next_flight = load_chunk(c + 1, 1 - slot) if c + 1 < NCHUNK else ()
        for d in in_flight:
            d.wait()
        if out_flight[slot] is not None:
            for d in out_flight[slot]:
                d.wait()
        descs = compute_chunk(c, slot)
        out_flight = (descs, out_flight[1]) if slot == 0 else (out_flight[0], descs)
        in_flight = next_flight
    for descs in out_flight:
        if descs is not None:
            for d in descs:
                d.wait()


@jax.jit
def kernel(disp_param, numbers, pad_mask, disp_param0):
    x = disp_param.reshape(-1)
    nums = numbers.reshape(-1)
    m32 = pad_mask.reshape(-1).view(jnp.uint8).view(jnp.int32)
    tab = jnp.concatenate(
        [disp_param0.reshape(-1), jnp.zeros((8,), jnp.float32)])

    mesh = plsc.VectorSubcoreMesh(
        core_axis_name="c", subcore_axis_name="s",
        num_cores=NC, num_subcores=NS)
    c6, al = pl.kernel(
        _body,
        out_type=[jax.ShapeDtypeStruct((N,), jnp.float32),
                  jax.ShapeDtypeStruct((N,), jnp.float32)],
        mesh=mesh,
        compiler_params=pltpu.CompilerParams(needs_layout_passes=False),
        scratch_types=[
            pltpu.VMEM((CH * 2,), jnp.float32),
            pltpu.VMEM((CH * 2,), jnp.float32),
            pltpu.VMEM((CH,), jnp.int32),
            pltpu.VMEM((CH,), jnp.int32),
            pltpu.VMEM((CH // 4,), jnp.int32),
            pltpu.VMEM((CH // 4,), jnp.int32),
            pltpu.VMEM((136,), jnp.float32),
            pltpu.VMEM((CH,), jnp.float32),
            pltpu.VMEM((CH,), jnp.float32),
            pltpu.VMEM((CH,), jnp.float32),
            pltpu.VMEM((CH,), jnp.float32),
            pltpu.SemaphoreType.DMA,
            pltpu.SemaphoreType.DMA,
            pltpu.SemaphoreType.DMA,
            pltpu.SemaphoreType.DMA,
        ],
    )(x, nums, m32, tab)
    return c6.reshape(B, S), al.reshape(B, S)


# speedup vs baseline: 9.6672x; 9.6672x over previous
import jax
import jax.numpy as jnp
from jax import lax
from jax.experimental import pallas as pl
from jax.experimental.pallas import tpu as pltpu
from jax.experimental.pallas import tpu_sc as plsc

NC = 2
NS = 16
NW = NC * NS
L = 16

B, S = 16384, 200
N = B * S
PER_W = N // NW
CH = 6400
NCHUNK = PER_W // CH

TC_ROWS = 64


def _tc_body(x0_ref, x1_ref, m_ref, c6_ref, al_ref):
    keep = jnp.logical_not(m_ref[...])
    c6_ref[...] = jnp.where(
        keep, jnp.exp(jnp.clip(x0_ref[...], -4.0, 4.0)), 0.0)
    al_ref[...] = jnp.where(
        keep, jnp.exp(jnp.clip(x1_ref[...], -4.0, 4.0)), 0.0)


def _sc_body(ec6_hbm, eal_hbm, nums_hbm, tab_hbm, c6_hbm, al_hbm,
             ec6_v0, ec6_v1, eal_v0, eal_v1, nums_v0, nums_v1, tab_v,
             oc6_v0, oc6_v1, oal_v0, oal_v1,
             sem_in0, sem_in1, sem_out0, sem_out1):
    wid = lax.axis_index("s") * NC + lax.axis_index("c")
    base = wid * PER_W
    ec6_v = (ec6_v0, ec6_v1)
    eal_v = (eal_v0, eal_v1)
    nums_v = (nums_v0, nums_v1)
    oc6_v = (oc6_v0, oc6_v1)
    oal_v = (oal_v0, oal_v1)
    sem_in = (sem_in0, sem_in1)
    sem_out = (sem_out0, sem_out1)

    pltpu.sync_copy(tab_hbm, tab_v)

    def load_chunk(c, slot):
        ebase = pl.multiple_of(base + c * CH, 64)
        return (
            pltpu.async_copy(ec6_hbm.at[pl.ds(ebase, CH)], ec6_v[slot],
                             sem_in[slot]),
            pltpu.async_copy(eal_hbm.at[pl.ds(ebase, CH)], eal_v[slot],
                             sem_in[slot]),
            pltpu.async_copy(nums_hbm.at[pl.ds(ebase, CH)], nums_v[slot],
                             sem_in[slot]),
        )

    def compute_chunk(c, slot):
        ec6s, eals, ns = ec6_v[slot], eal_v[slot], nums_v[slot]
        oc6, oal = oc6_v[slot], oal_v[slot]

        def step(t, _):
            nums16 = ns[pl.ds(t * L, L)]
            idx = nums16 * 2
            c6t = plsc.load_gather(tab_v, [idx])
            alt = plsc.load_gather(tab_v, [idx + 1])
            oc6[pl.ds(t * L, L)] = c6t * ec6s[pl.ds(t * L, L)]
            oal[pl.ds(t * L, L)] = alt * eals[pl.ds(t * L, L)]
            return _

        lax.fori_loop(0, CH // L, step, None)
        ebase = pl.multiple_of(base + c * CH, 64)
        return (
            pltpu.async_copy(oc6, c6_hbm.at[pl.ds(ebase, CH)], sem_out[slot]),
            pltpu.async_copy(oal, al_hbm.at[pl.ds(ebase, CH)], sem_out[slot]),
        )

    in_flight = load_chunk(0, 0)
    out_flight = (None, None)
    for c in range(NCHUNK):
        slot = c % 2
        next_flight = load_chunk(c + 1, 1 - slot) if c + 1 < NCHUNK else ()
        for d in in_flight:
            d.wait()
        if out_flight[slot] is not None:
            for d in out_flight[slot]:
                d.wait()
        descs = compute_chunk(c, slot)
        out_flight = (descs, out_flight[1]) if slot == 0 else (out_flight[0], descs)
        in_flight = next_flight
    for descs in out_flight:
        if descs is not None:
            for d in descs:
                d.wait()


@jax.jit
def kernel(disp_param, numbers, pad_mask, disp_param0):
    exc6, exal = pl.pallas_call(
        _tc_body,
        grid=(B // TC_ROWS,),
        in_specs=[
            pl.BlockSpec((TC_ROWS, S), lambda i: (i, 0)),
            pl.BlockSpec((TC_ROWS, S), lambda i: (i, 0)),
            pl.BlockSpec((TC_ROWS, S), lambda i: (i, 0)),
        ],
        out_specs=[
            pl.BlockSpec((TC_ROWS, S), lambda i: (i, 0)),
            pl.BlockSpec((TC_ROWS, S), lambda i: (i, 0)),
        ],
        out_shape=[jax.ShapeDtypeStruct((B, S), jnp.float32),
                   jax.ShapeDtypeStruct((B, S), jnp.float32)],
    )(disp_param[:, :, 0], disp_param[:, :, 1], pad_mask)

    nums = numbers.reshape(-1)
    tab = jnp.concatenate(
        [disp_param0.reshape(-1), jnp.zeros((8,), jnp.float32)])

    mesh = plsc.VectorSubcoreMesh(
        core_axis_name="c", subcore_axis_name="s",
        num_cores=NC, num_subcores=NS)
    c6, al = pl.kernel(
        _sc_body,
        out_type=[jax.ShapeDtypeStruct((N,), jnp.float32),
                  jax.ShapeDtypeStruct((N,), jnp.float32)],
        mesh=mesh,
        compiler_params=pltpu.CompilerParams(needs_layout_passes=False),
        scratch_types=[
            pltpu.VMEM((CH,), jnp.float32),
            pltpu.VMEM((CH,), jnp.float32),
            pltpu.VMEM((CH,), jnp.float32),
            pltpu.VMEM((CH,), jnp.float32),
            pltpu.VMEM((CH,), jnp.int32),
            pltpu.VMEM((CH,), jnp.int32),
            pltpu.VMEM((136,), jnp.float32),
            pltpu.VMEM((CH,), jnp.float32),
            pltpu.VMEM((CH,), jnp.float32),
            pltpu.VMEM((CH,), jnp.float32),
            pltpu.VMEM((CH,), jnp.float32),
            pltpu.SemaphoreType.DMA,
            pltpu.SemaphoreType.DMA,
            pltpu.SemaphoreType.DMA,
            pltpu.SemaphoreType.DMA,
        ],
    )(exc6.reshape(-1), exal.reshape(-1), nums, tab)
    return c6.reshape(B, S), al.reshape(B, S)


# speedup vs baseline: 12.1999x; 1.2620x over previous
import jax
import jax.numpy as jnp
from jax import lax
from jax.experimental import pallas as pl
from jax.experimental.pallas import tpu as pltpu
from jax.experimental.pallas import tpu_sc as plsc

NC = 2
NS = 16
NW = NC * NS
L = 16

B, S = 16384, 200
N = B * S
PER_W = N // NW
CH = 6400
NCHUNK = PER_W // CH

TC_ROWS = 64


TC_BLK = 131072


def _tc_body(x0_ref, x1_ref, m_ref, c6_ref, al_ref):
    keep = jnp.logical_not(m_ref[...])
    c6_ref[...] = jnp.where(
        keep, jnp.exp(jnp.clip(x0_ref[...], -4.0, 4.0)), 0.0)
    al_ref[...] = jnp.where(
        keep, jnp.exp(jnp.clip(x1_ref[...], -4.0, 4.0)), 0.0)


def _sc_body(ec6_hbm, eal_hbm, nums_hbm, tab_hbm, c6_hbm, al_hbm,
             ec6_v0, ec6_v1, eal_v0, eal_v1, nums_v0, nums_v1, tab_v,
             oc6_v0, oc6_v1, oal_v0, oal_v1,
             sem_in0, sem_in1, sem_out0, sem_out1):
    wid = lax.axis_index("s") * NC + lax.axis_index("c")
    base = wid * PER_W
    ec6_v = (ec6_v0, ec6_v1)
    eal_v = (eal_v0, eal_v1)
    nums_v = (nums_v0, nums_v1)
    oc6_v = (oc6_v0, oc6_v1)
    oal_v = (oal_v0, oal_v1)
    sem_in = (sem_in0, sem_in1)
    sem_out = (sem_out0, sem_out1)

    pltpu.sync_copy(tab_hbm, tab_v)

    def load_chunk(c, slot):
        ebase = pl.multiple_of(base + c * CH, 64)
        return (
            pltpu.async_copy(ec6_hbm.at[pl.ds(ebase, CH)], ec6_v[slot],
                             sem_in[slot]),
            pltpu.async_copy(eal_hbm.at[pl.ds(ebase, CH)], eal_v[slot],
                             sem_in[slot]),
            pltpu.async_copy(nums_hbm.at[pl.ds(ebase, CH)], nums_v[slot],
                             sem_in[slot]),
        )

    def compute_chunk(c, slot):
        ec6s, eals, ns = ec6_v[slot], eal_v[slot], nums_v[slot]
        oc6, oal = oc6_v[slot], oal_v[slot]

        def step(t, _):
            nums16 = ns[pl.ds(t * L, L)]
            idx = nums16 * 2
            c6t = plsc.load_gather(tab_v, [idx])
            alt = plsc.load_gather(tab_v, [idx + 1])
            oc6[pl.ds(t * L, L)] = c6t * ec6s[pl.ds(t * L, L)]
            oal[pl.ds(t * L, L)] = alt * eals[pl.ds(t * L, L)]
            return _

        lax.fori_loop(0, CH // L, step, None)
        ebase = pl.multiple_of(base + c * CH, 64)
        return (
            pltpu.async_copy(oc6, c6_hbm.at[pl.ds(ebase, CH)], sem_out[slot]),
            pltpu.async_copy(oal, al_hbm.at[pl.ds(ebase, CH)], sem_out[slot]),
        )

    in_flight = load_chunk(0, 0)
    out_flight = (None, None)
    for c in range(NCHUNK):
        slot = c % 2
        next_flight = load_chunk(c + 1, 1 - slot) if c + 1 < NCHUNK else ()
        for d in in_flight:
            d.wait()
        if out_flight[slot] is not None:
            for d in out_flight[slot]:
                d.wait()
        descs = compute_chunk(c, slot)
        out_flight = (descs, out_flight[1]) if slot == 0 else (out_flight[0], descs)
        in_flight = next_flight
    for descs in out_flight:
        if descs is not None:
            for d in descs:
                d.wait()


@jax.jit
def kernel(disp_param, numbers, pad_mask, disp_param0):
    exc6, exal = pl.pallas_call(
        _tc_body,
        grid=(N // TC_BLK,),
        in_specs=[
            pl.BlockSpec((TC_BLK,), lambda i: (i,)),
            pl.BlockSpec((TC_BLK,), lambda i: (i,)),
            pl.BlockSpec((TC_BLK,), lambda i: (i,)),
        ],
        out_specs=[
            pl.BlockSpec((TC_BLK,), lambda i: (i,)),
            pl.BlockSpec((TC_BLK,), lambda i: (i,)),
        ],
        out_shape=[jax.ShapeDtypeStruct((N,), jnp.float32),
                   jax.ShapeDtypeStruct((N,), jnp.float32)],
    )(disp_param[:, :, 0].reshape(-1), disp_param[:, :, 1].reshape(-1),
      pad_mask.reshape(-1))

    nums = numbers.reshape(-1)
    tab = jnp.concatenate(
        [disp_param0.reshape(-1), jnp.zeros((8,), jnp.float32)])

    mesh = plsc.VectorSubcoreMesh(
        core_axis_name="c", subcore_axis_name="s",
        num_cores=NC, num_subcores=NS)
    c6, al = pl.kernel(
        _sc_body,
        out_type=[jax.ShapeDtypeStruct((N,), jnp.float32),
                  jax.ShapeDtypeStruct((N,), jnp.float32)],
        mesh=mesh,
        compiler_params=pltpu.CompilerParams(needs_layout_passes=False),
        scratch_types=[
            pltpu.VMEM((CH,), jnp.float32),
            pltpu.VMEM((CH,), jnp.float32),
            pltpu.VMEM((CH,), jnp.float32),
            pltpu.VMEM((CH,), jnp.float32),
            pltpu.VMEM((CH,), jnp.int32),
            pltpu.VMEM((CH,), jnp.int32),
            pltpu.VMEM((136,), jnp.float32),
            pltpu.VMEM((CH,), jnp.float32),
            pltpu.VMEM((CH,), jnp.float32),
            pltpu.VMEM((CH,), jnp.float32),
            pltpu.VMEM((CH,), jnp.float32),
            pltpu.SemaphoreType.DMA,
            pltpu.SemaphoreType.DMA,
            pltpu.SemaphoreType.DMA,
            pltpu.SemaphoreType.DMA,
        ],
    )(exc6, exal, nums, tab)
    return c6.reshape(B, S), al.reshape(B, S)


# speedup vs baseline: 12.2206x; 1.0017x over previous
import jax
import jax.numpy as jnp
from jax import lax
from jax.experimental import pallas as pl
from jax.experimental.pallas import tpu as pltpu
from jax.experimental.pallas import tpu_sc as plsc

NC = 2
NS = 16
NW = NC * NS
L = 16

B, S = 16384, 200
N = B * S
PER_W = N // NW
CH = 6400
NCHUNK = PER_W // CH

TC_ROWS = 64


TC_BLK = 131072


def _tc_body(x0_ref, x1_ref, m_ref, c6_ref, al_ref):
    keep = jnp.logical_not(m_ref[...])
    c6_ref[...] = jnp.where(
        keep, jnp.exp(jnp.clip(x0_ref[...], -4.0, 4.0)), 0.0)
    al_ref[...] = jnp.where(
        keep, jnp.exp(jnp.clip(x1_ref[...], -4.0, 4.0)), 0.0)


def _sc_body(ec6_hbm, eal_hbm, nums_hbm, tab_hbm, c6_hbm, al_hbm,
             ec6_v0, ec6_v1, eal_v0, eal_v1, nums_v0, nums_v1, tab_v,
             oc6_v0, oc6_v1, oal_v0, oal_v1,
             sem_in0, sem_in1, sem_out0, sem_out1):
    wid = lax.axis_index("s") * NC + lax.axis_index("c")
    base = wid * PER_W
    ec6_v = (ec6_v0, ec6_v1)
    eal_v = (eal_v0, eal_v1)
    nums_v = (nums_v0, nums_v1)
    oc6_v = (oc6_v0, oc6_v1)
    oal_v = (oal_v0, oal_v1)
    sem_in = (sem_in0, sem_in1)
    sem_out = (sem_out0, sem_out1)

    pltpu.sync_copy(tab_hbm, tab_v)

    def load_chunk(c, slot):
        ebase = pl.multiple_of(base + c * CH, 64)
        return (
            pltpu.async_copy(ec6_hbm.at[pl.ds(ebase, CH)], ec6_v[slot],
                             sem_in[slot]),
            pltpu.async_copy(eal_hbm.at[pl.ds(ebase, CH)], eal_v[slot],
                             sem_in[slot]),
            pltpu.async_copy(nums_hbm.at[pl.ds(ebase, CH)], nums_v[slot],
                             sem_in[slot]),
        )

    def compute_chunk(c, slot):
        ec6s, eals, ns = ec6_v[slot], eal_v[slot], nums_v[slot]
        oc6, oal = oc6_v[slot], oal_v[slot]

        def step(g, _):
            for k in range(4):
                t = g * 4 + k
                nums16 = ns[pl.ds(t * L, L)]
                idx = nums16 * 2
                c6t = plsc.load_gather(tab_v, [idx])
                alt = plsc.load_gather(tab_v, [idx + 1])
                oc6[pl.ds(t * L, L)] = c6t * ec6s[pl.ds(t * L, L)]
                oal[pl.ds(t * L, L)] = alt * eals[pl.ds(t * L, L)]
            return _

        lax.fori_loop(0, CH // (4 * L), step, None)
        ebase = pl.multiple_of(base + c * CH, 64)
        return (
            pltpu.async_copy(oc6, c6_hbm.at[pl.ds(ebase, CH)], sem_out[slot]),
            pltpu.async_copy(oal, al_hbm.at[pl.ds(ebase, CH)], sem_out[slot]),
        )

    in_flight = load_chunk(0, 0)
    out_flight = (None, None)
    for c in range(NCHUNK):
        slot = c % 2
        next_flight = load_chunk(c + 1, 1 - slot) if c + 1 < NCHUNK else ()
        for d in in_flight:
            d.wait()
        if out_flight[slot] is not None:
            for d in out_flight[slot]:
                d.wait()
        descs = compute_chunk(c, slot)
        out_flight = (descs, out_flight[1]) if slot == 0 else (out_flight[0], descs)
        in_flight = next_flight
    for descs in out_flight:
        if descs is not None:
            for d in descs:
                d.wait()


@jax.jit
def kernel(disp_param, numbers, pad_mask, disp_param0):
    exc6, exal = pl.pallas_call(
        _tc_body,
        grid=(N // TC_BLK,),
        in_specs=[
            pl.BlockSpec((TC_BLK,), lambda i: (i,)),
            pl.BlockSpec((TC_BLK,), lambda i: (i,)),
            pl.BlockSpec((TC_BLK,), lambda i: (i,)),
        ],
        out_specs=[
            pl.BlockSpec((TC_BLK,), lambda i: (i,)),
            pl.BlockSpec((TC_BLK,), lambda i: (i,)),
        ],
        out_shape=[jax.ShapeDtypeStruct((N,), jnp.float32),
                   jax.ShapeDtypeStruct((N,), jnp.float32)],
    )(disp_param[:, :, 0].reshape(-1), disp_param[:, :, 1].reshape(-1),
      pad_mask.reshape(-1))

    nums = numbers.reshape(-1)
    tab = jnp.concatenate(
        [disp_param0.reshape(-1), jnp.zeros((8,), jnp.float32)])

    mesh = plsc.VectorSubcoreMesh(
        core_axis_name="c", subcore_axis_name="s",
        num_cores=NC, num_subcores=NS)
    c6, al = pl.kernel(
        _sc_body,
        out_type=[jax.ShapeDtypeStruct((N,), jnp.float32),
                  jax.ShapeDtypeStruct((N,), jnp.float32)],
        mesh=mesh,
        compiler_params=pltpu.CompilerParams(needs_layout_passes=False),
        scratch_types=[
            pltpu.VMEM((CH,), jnp.float32),
            pltpu.VMEM((CH,), jnp.float32),
            pltpu.VMEM((CH,), jnp.float32),
            pltpu.VMEM((CH,), jnp.float32),
            pltpu.VMEM((CH,), jnp.int32),
            pltpu.VMEM((CH,), jnp.int32),
            pltpu.VMEM((136,), jnp.float32),
            pltpu.VMEM((CH,), jnp.float32),
            pltpu.VMEM((CH,), jnp.float32),
            pltpu.VMEM((CH,), jnp.float32),
            pltpu.VMEM((CH,), jnp.float32),
            pltpu.SemaphoreType.DMA,
            pltpu.SemaphoreType.DMA,
            pltpu.SemaphoreType.DMA,
            pltpu.SemaphoreType.DMA,
        ],
    )(exc6, exal, nums, tab)
    return c6.reshape(B, S), al.reshape(B, S)
